# baseline (speedup 1.0000x reference)
import functools

import jax
import jax.numpy as jnp
from jax import lax
from jax.experimental import pallas as pl
from jax.experimental.pallas import tpu as pltpu
from jax.experimental.pallas import tpu_sc as plsc

P = 1024
K = 128
B = 16384

_CHUNK = 128
_BM = 2048


def _make_sc_gather(num_rows):
    info = plsc.get_sparse_core_info()
    nw = info.num_cores * info.num_subcores
    b_per_w = num_rows // nw
    n_chunks = b_per_w // _CHUNK
    mesh = plsc.VectorSubcoreMesh(core_axis_name="c", subcore_axis_name="s")

    @functools.partial(
        pl.kernel,
        mesh=mesh,
        out_type=jax.ShapeDtypeStruct((num_rows, K), jnp.float32),
        scratch_types=[
            pltpu.VMEM((n_chunks, _CHUNK), jnp.int32),
            pltpu.VMEM((b_per_w, K), jnp.float32),
            pltpu.SemaphoreType.DMA,
        ],
    )
    def gather_kernel(table_hbm, idx_hbm, out_hbm, idx_v, rows_v, sem):
        wid = lax.axis_index("s") * info.num_cores + lax.axis_index("c")
        base = wid * b_per_w
        pltpu.sync_copy(idx_hbm.at[wid], idx_v)
        copies = [
            pltpu.async_copy(
                table_hbm.at[idx_v.at[j]],
                rows_v.at[pl.ds(j * _CHUNK, _CHUNK)],
                sem,
            )
            for j in range(n_chunks)
        ]
        for c in copies:
            c.wait()
        pltpu.sync_copy(rows_v, out_hbm.at[pl.ds(base, b_per_w)])

    def run(table, idx):
        return gather_kernel(table, idx.reshape(nw, n_chunks, _CHUNK))

    return run


_sc_gather = _make_sc_gather(B)


def _matmul_body(zg_ref, w_ref, mu_ref, out_ref):
    acc = lax.dot_general(
        zg_ref[...],
        w_ref[...],
        dimension_numbers=(((1,), (1,)), ((), ())),
        preferred_element_type=jnp.float32,
    )
    out_ref[...] = acc + mu_ref[...]


_tc_matmul = pl.pallas_call(
    _matmul_body,
    grid=(B // _BM,),
    in_specs=[
        pl.BlockSpec((_BM, K), lambda i: (i, 0)),
        pl.BlockSpec((P, K), lambda i: (0, 0)),
        pl.BlockSpec((1, P), lambda i: (0, 0)),
    ],
    out_specs=pl.BlockSpec((_BM, P), lambda i: (i, 0)),
    out_shape=jax.ShapeDtypeStruct((B, P), jnp.float32),
    compiler_params=pltpu.CompilerParams(
        dimension_semantics=("arbitrary",),
    ),
)


def kernel(X, indices, z, W, mu):
    del X
    zg = _sc_gather(z, indices.astype(jnp.int32))
    return _tc_matmul(zg, W, mu.reshape(1, P))
